# baseline (speedup 1.0000x reference)
import functools

import jax
import jax.numpy as jnp
from jax import lax
from jax.experimental import pallas as pl
from jax.experimental.pallas import tpu as pltpu
from jax.experimental.pallas import tpu_sc as plsc

PAD_ROW = 0
V = 1000
H = 64
B = 1024
L = 50
LP = 56

NC = 2
NS = 16
NW = NC * NS
BPW = B // NW
TPW = BPW * LP


def _proj_body(emb_ref, w_ref, b_ref, out_ref):
    emb = emb_ref[:]
    rows = lax.broadcasted_iota(jnp.int32, emb.shape, 0)
    emb = jnp.where(rows == PAD_ROW, 0.0, emb)
    acc = lax.dot_general(
        emb, w_ref[:], (((1,), (1,)), ((), ())),
        preferred_element_type=jnp.float32,
    )
    out_ref[:] = acc + b_ref[:]


def _logit_table(emb, w, b):
    return pl.pallas_call(
        _proj_body,
        out_shape=jax.ShapeDtypeStruct((V, V), jnp.float32),
    )(emb, w, b.reshape(1, V))


def _gather_body(lt_hbm, ids_hbm, out_hbm, idx_v, a0, a1, a2, tail, lt_sh,
                 gsem, osem):
    c = lax.axis_index("c")
    s = lax.axis_index("s")
    wid = s * NC + c
    tok0 = wid * TPW
    b0 = wid * BPW

    row0 = jnp.where(s == NS - 1, V - 64, s * 64)
    pltpu.sync_copy(lt_hbm.at[pl.ds(row0, 64)], lt_sh.at[pl.ds(row0, 64)])
    pltpu.sync_copy(ids_hbm.at[pl.ds(tok0, TPW)], idx_v)
    plsc.subcore_barrier()

    bufs = (a0, a1, a2, tail)
    sizes = (16, 16, 16, 8)
    copy_rows = (16, 16, 16, 2)

    def step(i, carry):
        bb = b0 + i
        t0 = i * LP

        @pl.when(i >= 1)
        def _():
            for k in range(4):
                pltpu.make_async_copy(
                    bufs[k].at[pl.ds(0, copy_rows[k])],
                    out_hbm.at[bb - 1].at[pl.ds(16 * k, copy_rows[k])],
                    osem).wait()

        for k in range(4):
            pltpu.async_copy(
                lt_sh.at[idx_v.at[pl.ds(t0 + 16 * k, sizes[k])]],
                bufs[k], gsem).wait()
            pltpu.async_copy(
                bufs[k].at[pl.ds(0, copy_rows[k])],
                out_hbm.at[bb].at[pl.ds(16 * k, copy_rows[k])],
                osem)
        return carry

    lax.fori_loop(0, BPW, step, 0)
    for k in range(4):
        pltpu.make_async_copy(
            bufs[k].at[pl.ds(0, copy_rows[k])],
            out_hbm.at[b0 + BPW - 1].at[pl.ds(16 * k, copy_rows[k])],
            osem).wait()


_gather = functools.partial(
    pl.kernel,
    out_type=jax.ShapeDtypeStruct((B, L, V), jnp.float32),
    mesh=plsc.VectorSubcoreMesh(core_axis_name="c", subcore_axis_name="s"),
    compiler_params=pltpu.CompilerParams(use_tc_tiling_on_sc=False),
    scratch_types=[
        pltpu.VMEM((TPW,), jnp.int32),
        pltpu.VMEM((16, V), jnp.float32),
        pltpu.VMEM((16, V), jnp.float32),
        pltpu.VMEM((16, V), jnp.float32),
        pltpu.VMEM((8, V), jnp.float32),
        pltpu.VMEM_SHARED((V, V), jnp.float32),
        pltpu.SemaphoreType.DMA,
        pltpu.SemaphoreType.DMA,
    ],
)(_gather_body)


def kernel(token_ids, emb_table, W, b):
    lt = _logit_table(emb_table, W, b)
    ids_pad = jnp.pad(token_ids, ((0, 0), (0, LP - L))).reshape(-1)
    return _gather(lt, ids_pad)
